# baseline (speedup 1.0000x reference)
import functools

import jax
import jax.numpy as jnp
from jax import lax
from jax.experimental import pallas as pl
from jax.experimental.pallas import tpu as pltpu
from jax.experimental.pallas import tpu_sc as plsc

N = 3125
NP = 3328
B = 8
TASKS = 2 * B
NCH = NP // 16


def _sc_body(logit_hbm, lab_hbm, bbox_hbm, tgt_hbm, out_hbm, pout_hbm,
             labv, probv, x1v, y1v, x2v, y2v, tgtv,
             kc, ec, fc, ic, accv, pcov):
    c = lax.axis_index("c")
    s = lax.axis_index("s")
    wid = s * 2 + c
    task = wid // 2
    half = wid % 2
    b = task // 2
    l0 = (task % 2) == 0

    pltpu.sync_copy(lab_hbm.at[b], labv)
    pltpu.sync_copy(logit_hbm.at[b], probv)
    pltpu.sync_copy(bbox_hbm.at[b, 0], x1v)
    pltpu.sync_copy(bbox_hbm.at[b, 1], y1v)
    pltpu.sync_copy(bbox_hbm.at[b, 2], x2v)
    pltpu.sync_copy(bbox_hbm.at[b, 3], y2v)
    pltpu.sync_copy(tgt_hbm.at[b], tgtv)
    t16 = tgtv[...]
    tx1 = t16[0]
    ty1 = t16[1]
    tx2 = t16[2]
    ty2 = t16[3]
    ta = (tx2 - tx1) * (ty2 - ty1)

    iota = lax.iota(jnp.int32, 16)
    zero16 = jnp.zeros((16,), jnp.float32)

    def prob_loop(cj, mm):
        v0 = cj * 16
        pos = labv[pl.ds(v0, 16)] > 0.0
        prob = jnp.exp(probv[pl.ds(v0, 16)])
        probv[pl.ds(v0, 16)] = prob
        mn = jnp.minimum(mm[0], jnp.where(pos, prob, jnp.inf))
        mx = jnp.maximum(mm[1], jnp.where(pos, prob, -jnp.inf))
        return (mn, mx)

    mn16, mx16 = lax.fori_loop(0, NCH, prob_loop,
                               (jnp.full((16,), jnp.inf, jnp.float32),
                                jnp.full((16,), -jnp.inf, jnp.float32)))
    s1 = 0.5 * (jnp.min(mn16) + jnp.max(mx16))
    sh = jnp.where(l0, s1, 0.5)

    def comp_loop(cj, cnt):
        v0 = cj * 16
        pos = labv[pl.ds(v0, 16)] > 0.0
        x1 = x1v[pl.ds(v0, 16)]
        y1 = y1v[pl.ds(v0, 16)]
        x2 = x2v[pl.ds(v0, 16)]
        y2 = y2v[pl.ds(v0, 16)]
        ww = jnp.maximum(jnp.minimum(tx2, x2) - jnp.maximum(tx1, x1), 0.0)
        hh = jnp.maximum(jnp.minimum(ty2, y2) - jnp.maximum(ty1, y1), 0.0)
        inter = ww * hh
        iou = inter / ((x2 - x1) * (y2 - y1) + ta - inter)
        prob = probv[pl.ds(v0, 16)]
        key = jnp.where(l0, iou, prob)
        val = jnp.where(l0, prob, iou)
        ee = jnp.where(pos, jnp.exp(val - sh), 0.0)
        ff = jnp.where(pos, jnp.exp(sh - val), 0.0)
        cs = jnp.where(pos, 1, 0)
        for k in (1, 2, 4, 8):
            g = cs.at[jnp.maximum(iota - k, 0)].get(mode="promise_in_bounds")
            cs = cs + jnp.where(iota >= k, g, 0)
        idx = cnt + cs - 1
        plsc.store_scatter(kc, [idx], key, mask=pos)
        plsc.store_scatter(ec, [idx], ee, mask=pos)
        plsc.store_scatter(fc, [idx], ff, mask=pos)
        plsc.store_scatter(ic, [idx], v0 + iota, mask=pos)
        return cnt + cs[15]

    pc = lax.fori_loop(0, NCH, comp_loop, jnp.int32(0))
    kc[pl.ds(pc, 16)] = zero16
    ec[pl.ds(pc, 16)] = zero16
    fc[pl.ds(pc, 16)] = zero16

    nb = (pc + 15) // 16
    ncv = nb
    b0 = jnp.where(half == 0, 0, nb // 2)
    b1 = jnp.where(half == 0, nb // 2, nb)

    def u_loop(ub, acc):
        u0 = ub * 16
        ku16 = kc[pl.ds(u0, 16)]
        fu16 = fc[pl.ds(u0, 16)]
        iu16 = ic[pl.ds(u0, 16)]
        ee_d = ec[pl.ds(u0, 16)]
        for g in range(2):
            ku = [ku16[8 * g + j] for j in range(8)]
            fu = [fu16[8 * g + j] for j in range(8)]
            iu = [iu16[8 * g + j] for j in range(8)]

            @plsc.parallel_loop(0, ub, unroll=4, carry=(zero16,) * 8)
            def v_lt(cj, a):
                v0 = cj * 16
                kk = kc[pl.ds(v0, 16)]
                ee = ec[pl.ds(v0, 16)]
                return tuple(a[j] + jnp.where(kk < ku[j], ee, 0.0)
                             for j in range(8))

            @plsc.parallel_loop(ub + 1, ncv, unroll=4, carry=v_lt)
            def v_le(cj, a):
                v0 = cj * 16
                kk = kc[pl.ds(v0, 16)]
                ee = ec[pl.ds(v0, 16)]
                return tuple(a[j] + jnp.where(kk <= ku[j], ee, 0.0)
                             for j in range(8))

            a8 = v_le
            for j in range(8):
                cond = (ku16 < ku[j]) | ((ku16 == ku[j]) & (iu16 > iu[j]))
                av = a8[j] + jnp.where(cond, ee_d, 0.0)
                acc = acc + fu[j] * av
        return acc

    acc = lax.fori_loop(b0, b1, u_loop, zero16)
    accv[...] = acc
    pcov[...] = jnp.broadcast_to(jnp.float32(pc), (16,))
    pltpu.sync_copy(accv, out_hbm.at[wid])
    pltpu.sync_copy(pcov, pout_hbm.at[wid])


@jax.jit
def kernel(cls, label_cls, pred_bboxes, label_target):
    logit = cls.reshape(B, N, 2)[:, :, 1]
    logit = jnp.pad(logit, ((0, 0), (0, NP - N)))
    lab = jnp.pad(label_cls.reshape(B, N).astype(jnp.float32),
                  ((0, 0), (0, NP - N)))
    bbox = jnp.pad(pred_bboxes, ((0, 0), (0, 0), (0, NP - N)))
    tgt = jnp.pad(label_target, ((0, 0), (0, 12)))

    sc_call = functools.partial(
        pl.kernel,
        out_type=[jax.ShapeDtypeStruct((2 * TASKS, 16), jnp.float32),
                  jax.ShapeDtypeStruct((2 * TASKS, 16), jnp.float32)],
        mesh=plsc.VectorSubcoreMesh(core_axis_name="c", subcore_axis_name="s"),
        compiler_params=pltpu.CompilerParams(needs_layout_passes=False),
        scratch_types=[
            pltpu.VMEM((NP,), jnp.float32),
            pltpu.VMEM((NP,), jnp.float32),
            pltpu.VMEM((NP,), jnp.float32),
            pltpu.VMEM((NP,), jnp.float32),
            pltpu.VMEM((NP,), jnp.float32),
            pltpu.VMEM((NP,), jnp.float32),
            pltpu.VMEM((16,), jnp.float32),
            pltpu.VMEM((NP + 16,), jnp.float32),
            pltpu.VMEM((NP + 16,), jnp.float32),
            pltpu.VMEM((NP + 16,), jnp.float32),
            pltpu.VMEM((NP + 16,), jnp.int32),
            pltpu.VMEM((16,), jnp.float32),
            pltpu.VMEM((16,), jnp.float32),
        ],
    )(_sc_body)
    partials, pcout = sc_call(logit, lab, bbox, tgt)

    sums = jnp.sum(partials.reshape(B, 2, 2 * 16), axis=2)
    p = pcout[::4, 0]
    cnt = p * (p - 1.0) * 0.5
    loss1 = sums[:, 0] / cnt
    loss2 = sums[:, 1] / cnt
    valid = (p > 1.0) & ~jnp.isnan(loss1) & ~jnp.isnan(loss2)
    l1 = jnp.where(valid, loss1, 0.0)
    l2 = jnp.where(valid, loss2, 0.0)
    nvalid = jnp.sum(valid.astype(jnp.float32))
    final1 = jnp.where(nvalid > 0, jnp.sum(l1) / nvalid, 0.0)
    final2 = jnp.where(nvalid > 0, jnp.sum(l2) / nvalid, 0.0)
    return (final1, final2)
